# Initial kernel scaffold; baseline (speedup 1.0000x reference)
import jax
import jax.numpy as jnp
from jax.experimental import pallas as pl


def kernel(x, rW1, rb1, rW1g, rb1g, rW2, rb2, eW, eb):
    raise NotImplementedError("write your pallas kernel here")


# speedup vs baseline: 1.7400x; 1.7400x over previous
import functools
import math

import jax
import jax.numpy as jnp
from jax import lax
from jax.experimental import pallas as pl
from jax.experimental.pallas import tpu as pltpu

N, D, H, E, O = 4096, 2048, 128, 8, 1024
BK = 1024


def _router_body(x_ref, w1_ref, b1_ref, w1g_ref, b1g_ref, w2_ref, b2_ref,
                 out_ref):
    x = x_ref[...]
    h1 = lax.dot_general(x, w1_ref[...], (((1,), (1,)), ((), ())),
                         preferred_element_type=jnp.float32) + b1_ref[...]
    g = lax.dot_general(x, w1g_ref[...], (((1,), (1,)), ((), ())),
                        preferred_element_type=jnp.float32) + b1g_ref[...]
    h = jnp.maximum(h1 * jnp.maximum(g, 0.0), 0.0)
    logits = lax.dot_general(h, w2_ref[...], (((1,), (1,)), ((), ())),
                             preferred_element_type=jnp.float32) + b2_ref[...]
    out_ref[...] = logits.T


def _router(x, rW1, rb1, rW1g, rb1g, rW2, rb2, interpret=False):
    TN = 512
    return pl.pallas_call(
        _router_body,
        grid=(N // TN,),
        in_specs=[
            pl.BlockSpec((TN, D), lambda i: (i, 0)),
            pl.BlockSpec((H, D), lambda i: (0, 0)),
            pl.BlockSpec((H,), lambda i: (0,)),
            pl.BlockSpec((H, D), lambda i: (0, 0)),
            pl.BlockSpec((H,), lambda i: (0,)),
            pl.BlockSpec((E, H), lambda i: (0, 0)),
            pl.BlockSpec((E,), lambda i: (0,)),
        ],
        out_specs=pl.BlockSpec((E, TN), lambda i: (0, i)),
        out_shape=jax.ShapeDtypeStruct((E, N), jnp.float32),
        interpret=interpret,
    )(x, rW1, rb1, rW1g, rb1g, rW2, rb2)


def _cumsum_lanes(v):
    r = v
    k = 1
    while k < N:
        shifted = jnp.concatenate(
            [jnp.zeros((E, k), r.dtype), r[:, : N - k]], axis=1)
        r = r + shifted
        k *= 2
    return r


def _select_body(lt_ref, posm_ref, wt_ref):
    lt = lt_ref[...]
    m0 = jnp.max(lt, axis=1, keepdims=True)
    p = jnp.exp(lt - m0)
    l = p / jnp.sum(p, axis=1, keepdims=True)
    bits = lax.bitcast_convert_type(l, jnp.int32)

    def step(_, carry):
        lo, hi = carry
        mid = lo + lax.shift_right_logical(hi - lo + 1, 1)
        cnt = jnp.sum((bits >= mid).astype(jnp.int32), axis=1, keepdims=True)
        ok = cnt >= BK
        return jnp.where(ok, mid, lo), jnp.where(ok, hi, mid - 1)

    lo0 = jnp.zeros((E, 1), jnp.int32)
    hi0 = jnp.full((E, 1), 0x7F800000, jnp.int32)
    lo, _ = lax.fori_loop(0, 31, step, (lo0, hi0))
    thr = lo

    sel_gt = bits > thr
    tie = bits == thr
    n_gt = jnp.sum(sel_gt.astype(jnp.int32), axis=1, keepdims=True)
    need = BK - n_gt
    tie_rank = _cumsum_lanes(tie.astype(jnp.int32))
    sel = sel_gt | (tie & (tie_rank <= need))

    m1 = jnp.max(l, axis=1, keepdims=True)
    num = jnp.where(sel, jnp.exp(l - m1), 0.0)
    den = jnp.sum(num, axis=1, keepdims=True)
    wt_ref[...] = num / den

    pos = _cumsum_lanes(sel.astype(jnp.int32)) - 1
    posm_ref[...] = jnp.where(sel, pos, -1)


def _select(logitsT, interpret=False):
    return pl.pallas_call(
        _select_body,
        out_shape=(
            jax.ShapeDtypeStruct((E, N), jnp.int32),
            jax.ShapeDtypeStruct((E, N), jnp.float32),
        ),
        interpret=interpret,
    )(logitsT)


def _expert_body(xg_ref, w_ref, b_ref, ws_ref, out_ref):
    t = pl.program_id(1)
    tb = xg_ref.shape[0]
    acc = lax.dot_general(xg_ref[...], w_ref[0], (((1,), (1,)), ((), ())),
                          preferred_element_type=jnp.float32)
    ws = ws_ref[0, 0, pl.ds(t * tb, tb)]
    out_ref[0] = (acc + b_ref[0, 0]) * ws[:, None]


def _experts(xg, eW, eb3, wsel4, interpret=False):
    TB = 256
    return pl.pallas_call(
        _expert_body,
        grid=(E, BK // TB),
        in_specs=[
            pl.BlockSpec((TB, D), lambda e, t: (e * (BK // TB) + t, 0)),
            pl.BlockSpec((1, O, D), lambda e, t: (e, 0, 0)),
            pl.BlockSpec((1, 1, O), lambda e, t: (e, 0, 0)),
            pl.BlockSpec((1, 1, BK), lambda e, t: (e, 0, 0)),
        ],
        out_specs=pl.BlockSpec((1, TB, O), lambda e, t: (e, t, 0)),
        out_shape=jax.ShapeDtypeStruct((E, BK, O), jnp.float32),
        compiler_params=pltpu.CompilerParams(
            dimension_semantics=("arbitrary", "arbitrary")),
        interpret=interpret,
    )(xg, eW, eb3, wsel4)


def _combine_body(posm_ref, *refs):
    ys_refs = refs[:E]
    out_ref = refs[E]
    sel = posm_ref[...] >= 0
    cols = []
    ts = jnp.zeros_like(ys_refs[0][...])
    for e in range(E):
        m = sel[:, e:e + 1]
        ye = jnp.where(m, ys_refs[e][...], 0.0)
        cols.append(ye)
        ts = ts + ye
    out_ref[...] = jnp.concatenate([c + ts for c in cols], axis=1)


def _combine(posmN, ys_list, interpret=False):
    TN = 256
    return pl.pallas_call(
        _combine_body,
        grid=(N // TN,),
        in_specs=[pl.BlockSpec((TN, E), lambda i: (i, 0))] +
                 [pl.BlockSpec((TN, O), lambda i: (i, 0)) for _ in range(E)],
        out_specs=pl.BlockSpec((TN, E * O), lambda i: (i, 0)),
        out_shape=jax.ShapeDtypeStruct((N, E * O), jnp.float32),
        interpret=interpret,
    )(posmN, *ys_list)


def _moe(x, rW1, rb1, rW1g, rb1g, rW2, rb2, eW, eb, interpret=False):
    logitsT = _router(x, rW1, rb1, rW1g, rb1g, rW2, rb2, interpret=interpret)
    posmT, WT = _select(logitsT, interpret=interpret)

    big = jnp.where(posmT >= 0, posmT, N)
    ib = jnp.argsort(big, axis=1)[:, :BK].astype(jnp.int32)
    wsel = jnp.take_along_axis(WT, ib, axis=1)
    xg = x[ib.reshape(-1)]

    ye = _experts(xg, eW, eb.reshape(E, 1, O), wsel.reshape(E, 1, BK),
                  interpret=interpret)

    ys_list = [
        jnp.zeros((N, O), jnp.float32).at[ib[e]].set(ye[e]) for e in range(E)
    ]
    return _combine(posmT.T, ys_list, interpret=interpret)


def kernel(x, rW1, rb1, rW1g, rb1g, rW2, rb2, eW, eb):
    return _moe(x, rW1, rb1, rW1g, rb1g, rW2, rb2, eW, eb)


# speedup vs baseline: 2.3582x; 1.3553x over previous
import functools
import math

import jax
import jax.numpy as jnp
from jax import lax
from jax.experimental import pallas as pl
from jax.experimental.pallas import tpu as pltpu

N, D, H, E, O = 4096, 2048, 128, 8, 1024
BK = 1024


def _router_body(x_ref, w1_ref, b1_ref, w1g_ref, b1g_ref, w2_ref, b2_ref,
                 out_ref):
    x = x_ref[...]
    h1 = lax.dot_general(x, w1_ref[...], (((1,), (1,)), ((), ())),
                         preferred_element_type=jnp.float32) + b1_ref[...]
    g = lax.dot_general(x, w1g_ref[...], (((1,), (1,)), ((), ())),
                        preferred_element_type=jnp.float32) + b1g_ref[...]
    h = jnp.maximum(h1 * jnp.maximum(g, 0.0), 0.0)
    logits = lax.dot_general(h, w2_ref[...], (((1,), (1,)), ((), ())),
                             preferred_element_type=jnp.float32) + b2_ref[...]
    out_ref[...] = logits.T


def _router(x, rW1, rb1, rW1g, rb1g, rW2, rb2, interpret=False):
    TN = 512
    return pl.pallas_call(
        _router_body,
        grid=(N // TN,),
        in_specs=[
            pl.BlockSpec((TN, D), lambda i: (i, 0)),
            pl.BlockSpec((H, D), lambda i: (0, 0)),
            pl.BlockSpec((H,), lambda i: (0,)),
            pl.BlockSpec((H, D), lambda i: (0, 0)),
            pl.BlockSpec((H,), lambda i: (0,)),
            pl.BlockSpec((E, H), lambda i: (0, 0)),
            pl.BlockSpec((E,), lambda i: (0,)),
        ],
        out_specs=pl.BlockSpec((E, TN), lambda i: (0, i)),
        out_shape=jax.ShapeDtypeStruct((E, N), jnp.float32),
        interpret=interpret,
    )(x, rW1, rb1, rW1g, rb1g, rW2, rb2)


def _cumsum_lanes(v):
    r = v
    k = 1
    while k < N:
        shifted = jnp.concatenate(
            [jnp.zeros((E, k), r.dtype), r[:, : N - k]], axis=1)
        r = r + shifted
        k *= 2
    return r


def _select_body(lt_ref, posm_ref, wt_ref):
    lt = lt_ref[...]
    m0 = jnp.max(lt, axis=1, keepdims=True)
    p = jnp.exp(lt - m0)
    l = p / jnp.sum(p, axis=1, keepdims=True)
    bits = lax.bitcast_convert_type(l, jnp.int32)

    def step(_, carry):
        lo, hi = carry
        mid = lo + lax.shift_right_logical(hi - lo + 1, 1)
        cnt = jnp.sum((bits >= mid).astype(jnp.int32), axis=1, keepdims=True)
        ok = cnt >= BK
        return jnp.where(ok, mid, lo), jnp.where(ok, hi, mid - 1)

    lo0 = jnp.zeros((E, 1), jnp.int32)
    hi0 = jnp.full((E, 1), 0x7F800000, jnp.int32)
    lo, _ = lax.fori_loop(0, 31, step, (lo0, hi0))
    thr = lo

    sel_gt = bits > thr
    tie = bits == thr
    n_gt = jnp.sum(sel_gt.astype(jnp.int32), axis=1, keepdims=True)
    need = BK - n_gt
    tie_rank = _cumsum_lanes(tie.astype(jnp.int32))
    sel = sel_gt | (tie & (tie_rank <= need))

    m1 = jnp.max(l, axis=1, keepdims=True)
    num = jnp.where(sel, jnp.exp(l - m1), 0.0)
    den = jnp.sum(num, axis=1, keepdims=True)
    wt_ref[...] = num / den

    pos = _cumsum_lanes(sel.astype(jnp.int32)) - 1
    posm_ref[...] = jnp.where(sel, pos, -1)


def _select(logitsT, interpret=False):
    return pl.pallas_call(
        _select_body,
        out_shape=(
            jax.ShapeDtypeStruct((E, N), jnp.int32),
            jax.ShapeDtypeStruct((E, N), jnp.float32),
        ),
        interpret=interpret,
    )(logitsT)


def _compact_sc(posmT, WT):
    from jax.experimental.pallas import tpu_sc as plsc
    info = plsc.get_sparse_core_info()
    NC, NS, L = info.num_cores, info.num_subcores, info.num_lanes
    mesh = plsc.VectorSubcoreMesh(core_axis_name="c", subcore_axis_name="s")

    @functools.partial(
        pl.kernel, mesh=mesh,
        compiler_params=pltpu.CompilerParams(needs_layout_passes=False),
        out_type=(
            jax.ShapeDtypeStruct((E, BK), jnp.int32),
            jax.ShapeDtypeStruct((E, BK), jnp.float32),
        ),
        scratch_types=[
            pltpu.VMEM((N,), jnp.int32),
            pltpu.VMEM((N,), jnp.float32),
            pltpu.VMEM((BK,), jnp.int32),
            pltpu.VMEM((BK,), jnp.float32),
        ],
    )
    def k(posm_hbm, wt_hbm, ib_hbm, ws_hbm, posv, wv, ibuf, wbuf):
        wid = lax.axis_index("s") * NC + lax.axis_index("c")

        @pl.when(wid < E)
        def _():
            pltpu.sync_copy(posm_hbm.at[wid], posv)
            pltpu.sync_copy(wt_hbm.at[wid], wv)

            def chunk(c, carry):
                p = posv[pl.ds(c * L, L)]
                w = wv[pl.ds(c * L, L)]
                mask = p >= 0
                tok = lax.iota(jnp.int32, L) + c * L
                plsc.store_scatter(ibuf, [p], tok, mask=mask)
                plsc.store_scatter(wbuf, [p], w, mask=mask)
                return carry

            lax.fori_loop(0, N // L, chunk, 0)
            pltpu.sync_copy(ibuf, ib_hbm.at[wid])
            pltpu.sync_copy(wbuf, ws_hbm.at[wid])

    return k(posmT, WT)


def _gather_sc(x, ib):
    from jax.experimental.pallas import tpu_sc as plsc
    info = plsc.get_sparse_core_info()
    NC, NS, L = info.num_cores, info.num_subcores, info.num_lanes
    NW = NC * NS
    RPW = (E * BK) // NW
    CH = 16
    mesh = plsc.VectorSubcoreMesh(core_axis_name="c", subcore_axis_name="s")

    NCHUNK = RPW // CH

    @functools.partial(
        pl.kernel, mesh=mesh,
        compiler_params=pltpu.CompilerParams(needs_layout_passes=False),
        out_type=jax.ShapeDtypeStruct((E * BK, D), jnp.float32),
        scratch_types=[
            pltpu.VMEM((2, CH), jnp.int32),
            pltpu.VMEM((CH, D), jnp.float32),
            pltpu.VMEM((CH, D), jnp.float32),
            pltpu.SemaphoreType.DMA,
            pltpu.SemaphoreType.DMA,
            pltpu.SemaphoreType.DMA,
            pltpu.SemaphoreType.DMA,
        ],
    )
    def k(x_hbm, ib_hbm, xg_hbm, idxv, rows0, rows1, g0, g1, w0, w1):
        wid = lax.axis_index("s") * NC + lax.axis_index("c")
        e = wid // (BK // RPW)
        q = wid % (BK // RPW)
        rows = (rows0, rows1)
        gsem = (g0, g1)
        wsem = (w0, w1)

        gd = [None, None]
        wd = [None, None]
        for c in range(NCHUNK):
            b = c % 2
            off = q * RPW + c * CH
            pltpu.sync_copy(ib_hbm.at[e, pl.ds(off, CH)], idxv.at[b])
            if wd[b] is not None:
                wd[b].wait()
            gd[b] = pltpu.async_copy(x_hbm.at[idxv.at[b]], rows[b], gsem[b])
            if c >= 1:
                pb = 1 - b
                poff = q * RPW + (c - 1) * CH
                gd[pb].wait()
                wd[pb] = pltpu.async_copy(
                    rows[pb], xg_hbm.at[pl.ds(e * BK + poff, CH), :],
                    wsem[pb])
        lb = (NCHUNK - 1) % 2
        loff = q * RPW + (NCHUNK - 1) * CH
        gd[lb].wait()
        wd[lb] = pltpu.async_copy(
            rows[lb], xg_hbm.at[pl.ds(e * BK + loff, CH), :], wsem[lb])
        wd[lb].wait()
        wd[1 - lb].wait()

    return k(x, ib)


def _scatter_sc(ye, ib):
    from jax.experimental.pallas import tpu_sc as plsc
    info = plsc.get_sparse_core_info()
    NC, NS, L = info.num_cores, info.num_subcores, info.num_lanes
    NW = NC * NS
    QPE = NW // E
    RPW = BK // QPE
    CH = 16
    mesh = plsc.VectorSubcoreMesh(core_axis_name="c", subcore_axis_name="s")

    @functools.partial(
        pl.kernel, mesh=mesh,
        compiler_params=pltpu.CompilerParams(needs_layout_passes=False),
        out_type=tuple(jax.ShapeDtypeStruct((N, O), jnp.float32)
                       for _ in range(E)),
        scratch_types=[
            pltpu.VMEM((2, CH), jnp.int32),
            pltpu.VMEM((CH, O), jnp.float32),
            pltpu.VMEM((CH, O), jnp.float32),
            pltpu.SemaphoreType.DMA,
            pltpu.SemaphoreType.DMA,
            pltpu.SemaphoreType.DMA,
            pltpu.SemaphoreType.DMA,
        ],
    )
    def k(ye_hbm, ib_hbm, *rest):
        ys_refs = rest[:E]
        idxv, rowsa, rowsb, l0, l1, s0, s1 = rest[E:]
        wid = lax.axis_index("s") * NC + lax.axis_index("c")
        eid = wid // QPE
        q = wid % QPE
        rows = (rowsa, rowsb)
        lsem = (l0, l1)
        ssem = (s0, s1)
        NCHUNK = RPW // CH

        for e in range(E):
            @pl.when(eid == e)
            def _(e=e):
                ld = [None, None]
                sd = [None, None]
                for c in range(NCHUNK):
                    b = c % 2
                    off = q * RPW + c * CH
                    if sd[b] is not None:
                        sd[b].wait()
                    pltpu.sync_copy(ib_hbm.at[e, pl.ds(off, CH)], idxv.at[b])
                    ld[b] = pltpu.async_copy(
                        ye_hbm.at[e, pl.ds(off, CH), :], rows[b], lsem[b])
                    if c >= 1:
                        pb = 1 - b
                        ld[pb].wait()
                        sd[pb] = pltpu.async_copy(
                            rows[pb], ys_refs[e].at[idxv.at[pb]], ssem[pb])
                lb = (NCHUNK - 1) % 2
                ld[lb].wait()
                sd[lb] = pltpu.async_copy(
                    rows[lb], ys_refs[e].at[idxv.at[lb]], ssem[lb])
                sd[lb].wait()
                sd[1 - lb].wait()

    return k(ye, ib)


def _expert_body(xg_ref, w_ref, b_ref, ws_ref, out_ref):
    t = pl.program_id(1)
    tb = xg_ref.shape[0]
    acc = lax.dot_general(xg_ref[...].astype(jnp.bfloat16),
                          w_ref[0].astype(jnp.bfloat16),
                          (((1,), (1,)), ((), ())),
                          preferred_element_type=jnp.float32)
    ws = ws_ref[0, 0, pl.ds(t * tb, tb)]
    out_ref[0] = (acc + b_ref[0, 0]) * ws[:, None]


def _experts(xg, eW, eb3, wsel4, interpret=False):
    TB = 256
    return pl.pallas_call(
        _expert_body,
        grid=(E, BK // TB),
        in_specs=[
            pl.BlockSpec((TB, D), lambda e, t: (e * (BK // TB) + t, 0)),
            pl.BlockSpec((1, O, D), lambda e, t: (e, 0, 0)),
            pl.BlockSpec((1, 1, O), lambda e, t: (e, 0, 0)),
            pl.BlockSpec((1, 1, BK), lambda e, t: (e, 0, 0)),
        ],
        out_specs=pl.BlockSpec((1, TB, O), lambda e, t: (e, t, 0)),
        out_shape=jax.ShapeDtypeStruct((E, BK, O), jnp.float32),
        compiler_params=pltpu.CompilerParams(
            dimension_semantics=("arbitrary", "arbitrary")),
        interpret=interpret,
    )(xg, eW, eb3, wsel4)


def _combine_body(posm_ref, *refs):
    ys_refs = refs[:E]
    out_ref = refs[E]
    sel = posm_ref[...] >= 0
    cols = []
    ts = jnp.zeros_like(ys_refs[0][...])
    for e in range(E):
        m = sel[:, e:e + 1]
        ye = jnp.where(m, ys_refs[e][...], 0.0)
        cols.append(ye)
        ts = ts + ye
    out_ref[...] = jnp.concatenate([c + ts for c in cols], axis=1)


def _combine(posmN, ys_list, interpret=False):
    TN = 256
    return pl.pallas_call(
        _combine_body,
        grid=(N // TN,),
        in_specs=[pl.BlockSpec((TN, E), lambda i: (i, 0))] +
                 [pl.BlockSpec((TN, O), lambda i: (i, 0)) for _ in range(E)],
        out_specs=pl.BlockSpec((TN, E * O), lambda i: (i, 0)),
        out_shape=jax.ShapeDtypeStruct((N, E * O), jnp.float32),
        interpret=interpret,
    )(posmN, *ys_list)


def _moe(x, rW1, rb1, rW1g, rb1g, rW2, rb2, eW, eb, interpret=False):
    logitsT = _router(x, rW1, rb1, rW1g, rb1g, rW2, rb2, interpret=interpret)
    posmT, WT = _select(logitsT, interpret=interpret)

    ib, wsel = _compact_sc(posmT, WT)
    xg = _gather_sc(x, ib)

    ye = _experts(xg, eW, eb.reshape(E, 1, O), wsel.reshape(E, 1, BK),
                  interpret=interpret)

    ys_list = _scatter_sc(ye, ib)
    return _combine(posmT.T, list(ys_list), interpret=interpret)


def kernel(x, rW1, rb1, rW1g, rb1g, rW2, rb2, eW, eb):
    return _moe(x, rW1, rb1, rW1g, rb1g, rW2, rb2, eW, eb)
